# baseline (speedup 1.0000x reference)
import functools

import jax
import jax.numpy as jnp
from jax import lax
from jax.experimental import pallas as pl
from jax.experimental.pallas import tpu as pltpu
from jax.experimental.pallas import tpu_sc as plsc

D = 64
SCALE = 8.0
NC, NS, L = 2, 16, 16
NW = NC * NS
NTOK = 4096
NSEQ = 200
VOC = 1000000
CH = 128
NCH = NSEQ * (NTOK // CH)
CPW = NCH // NW
CBLK = NTOK // CH
OUT_FLAT = NSEQ * 8 * CBLK * 8 * 128


def _sc_embed(table128, tok4):
    mesh = plsc.VectorSubcoreMesh(
        core_axis_name="c", subcore_axis_name="s", num_cores=NC, num_subcores=NS
    )

    @functools.partial(
        pl.kernel,
        out_type=jax.ShapeDtypeStruct((OUT_FLAT // 1024, 8, 128), jnp.float32),
        mesh=mesh,
        compiler_params=pltpu.CompilerParams(
            use_tc_tiling_on_sc=False, needs_layout_passes=False
        ),
        scratch_types=[
            pltpu.VMEM((CH,), jnp.int32),
            pltpu.VMEM((CH,), jnp.int32),
            pltpu.VMEM((CH, 128), jnp.float32),
            pltpu.VMEM((CH, 128), jnp.float32),
            pltpu.VMEM((64, 129), jnp.float32),
            pltpu.VMEM((64, 129), jnp.float32),
            pltpu.SemaphoreType.DMA,
            pltpu.SemaphoreType.DMA,
            pltpu.SemaphoreType.DMA,
            pltpu.SemaphoreType.DMA,
        ],
    )
    def k(tab_hbm, tok_hbm, out_hbm,
          idx0, idx1, rows0, rows1, st0, st1, g0, g1, s0, s1):
        wid = lax.axis_index("s") * NC + lax.axis_index("c")
        ch_base = wid * CPW
        lanes = lax.iota(jnp.int32, L)

        def load_idx(ch, idxv):
            j = ch // CBLK
            cb = ch % CBLK
            pltpu.sync_copy(tok_hbm.at[j // 8, cb, j % 8], idxv)

        def start_gather(idxv, rowsv, sem):
            pltpu.async_copy(tab_hbm.at[idxv], rowsv, sem)

        def wait_gather(rowsv, sem):
            pltpu.make_async_copy(tab_hbm.at[pl.ds(0, CH)], rowsv, sem).wait()

        def assemble(rowsv, stv):
            @plsc.parallel_loop(0, CH, unroll=8)
            def _(q):
                qv = jnp.broadcast_to(q, (L,))
                for kk in range(D // L):
                    vec = rowsv.at[q, pl.ds(L * kk, L)][...] * SCALE
                    plsc.store_scatter(stv, [lanes + L * kk, qv], vec)

        def start_stores(stv, ch, sem):
            j = ch // CBLK
            cb = ch % CBLK
            base = j * 8 * CBLK + cb
            for t in range(8):
                pltpu.async_copy(stv.at[pl.ds(8 * t, 8), pl.ds(0, 128)],
                                 out_hbm.at[base + CBLK * t],
                                 sem)

        def wait_stores(stv, sem):
            for t in range(8):
                pltpu.make_async_copy(stv.at[pl.ds(0, 8), pl.ds(0, 128)],
                                      out_hbm.at[0],
                                      sem).wait()

        load_idx(ch_base, idx0)
        start_gather(idx0, rows0, g0)

        @pl.loop(0, CPW, step=2)
        def _(kk):
            ch0 = ch_base + kk
            load_idx(ch0 + 1, idx1)
            start_gather(idx1, rows1, g1)
            wait_gather(rows0, g0)

            @pl.when(kk > 0)
            def _():
                wait_stores(st0, s0)

            assemble(rows0, st0)
            start_stores(st0, ch0, s0)

            @pl.when(kk + 2 < CPW)
            def _():
                load_idx(ch0 + 2, idx0)
                start_gather(idx0, rows0, g0)

            wait_gather(rows1, g1)

            @pl.when(kk > 0)
            def _():
                wait_stores(st1, s1)

            assemble(rows1, st1)
            start_stores(st1, ch0 + 1, s1)

        wait_stores(st0, s0)
        wait_stores(st1, s1)

    return k(table128, tok4)


def kernel(tokens, embedding):
    table128 = jnp.concatenate([embedding, embedding], axis=1)
    tok4 = (tokens.astype(jnp.int32).T
            .reshape(NSEQ // 8, 8, NTOK // 128, 128)
            .transpose(0, 2, 1, 3))
    out3 = _sc_embed(table128, tok4)
    out5 = out3.reshape(NSEQ, 8, CBLK, 8, 128)
    return out5.transpose(2, 4, 0, 1, 3).reshape(NTOK, NSEQ, D)
